# baseline (speedup 1.0000x reference)
import functools

import jax
import jax.numpy as jnp
from jax.experimental import pallas as pl
from jax.experimental.pallas import tpu as pltpu

F32 = jnp.float32
BF16 = jnp.bfloat16
_VMEM_LIMIT = 63 * 1024 * 1024
_INTERPRET = False


def _cparams(n):
    return pltpu.CompilerParams(
        dimension_semantics=("arbitrary",) * n,
        vmem_limit_bytes=_VMEM_LIMIT,
    )


def _gemv_trans_kernel(h_ref, w_ref, z_ref, wt_ref, wtb_ref, *eye_ref,
                       relu, scale, eye_b):
    h = h_ref[...]
    if relu:
        h = jnp.maximum(h, 0.0)
    z_ref[...] = jax.lax.dot_general(
        h, w_ref[...], (((1,), (1,)), ((), ())),
        preferred_element_type=F32)
    w = w_ref[...]
    wt_ref[...] = w.T
    if scale:
        w = w * (h_ref[...] > 0).astype(F32)
    wtb_ref[...] = w.T.astype(BF16)
    if eye_b:
        jj = pl.program_id(0)
        rows = jax.lax.broadcasted_iota(jnp.int32, (2048, eye_b), 0)
        cols = jax.lax.broadcasted_iota(jnp.int32, (2048, eye_b), 1) + jj * eye_b
        eye_ref[0][...] = jnp.where(rows == cols, 1.0, 0.0).astype(F32)


def _gemv_trans(h, W, bj, relu, scale, eye=False):
    J, K = W.shape
    eye_b = (2048 * bj) // J if eye else 0
    out_specs = [pl.BlockSpec((1, bj), lambda j: (0, j)),
                 pl.BlockSpec((K, bj), lambda j: (0, j)),
                 pl.BlockSpec((K, bj), lambda j: (0, j))]
    out_shape = [jax.ShapeDtypeStruct((1, J), F32),
                 jax.ShapeDtypeStruct((K, J), F32),
                 jax.ShapeDtypeStruct((K, J), BF16)]
    if eye:
        out_specs.append(pl.BlockSpec((2048, eye_b), lambda j: (0, j)))
        out_shape.append(jax.ShapeDtypeStruct((2048, 2048), F32))
    return pl.pallas_call(
        functools.partial(_gemv_trans_kernel, relu=relu, scale=scale,
                          eye_b=eye_b),
        grid=(J // bj,),
        in_specs=[pl.BlockSpec((1, K), lambda j: (0, 0)),
                  pl.BlockSpec((bj, K), lambda j: (j, 0))],
        out_specs=out_specs,
        out_shape=out_shape,
        compiler_params=_cparams(1),
        name="gemv_trans",
        interpret=_INTERPRET,
    )(h, W)


def _mmf_kernel(a_ref, b2_ref, b3_ref, z1_ref, z2_ref,
                d1_ref, d2_ref, djm_ref, t1s_ref, *, bj):
    j = pl.program_id(0)

    @pl.when(j < 32)
    def _():
        i = j // 16
        jj = j - i * 16
        o = jnp.dot(a_ref[...], b2_ref[...], preferred_element_type=F32)
        m2 = (z2_ref[...] > 0).astype(F32)
        r0 = pl.multiple_of(i * 1024, 1024)
        c0 = pl.multiple_of(jj * bj, bj)
        t1s_ref[pl.ds(r0, 1024), pl.ds(c0, bj)] = (o * m2).astype(BF16)

        d1_ref[...] = jnp.zeros((2048, bj), F32)
        d2_ref[...] = jnp.zeros((2048, bj), F32)

        @pl.when(jj // 8 == i)
        def _():
            s0 = pl.multiple_of((jj - i * 8) * bj, bj)
            rs = jax.lax.broadcasted_iota(jnp.int32, (bj, bj), 0)
            cs = jax.lax.broadcasted_iota(jnp.int32, (bj, bj), 1)
            eqs = rs == cs
            d1_ref[pl.ds(s0, bj), :] = jnp.where(
                eqs, (z1_ref[...] > 0).astype(F32), 0.0)
            d2_ref[pl.ds(s0, bj), :] = jnp.where(eqs, m2, 0.0)

    @pl.when(j >= 32)
    def _():
        djm_ref[...] = jnp.dot(t1s_ref[...], b3_ref[...],
                               preferred_element_type=F32)


def _mm_fused(A, B2, B3, z1, z2, bj=256):
    d_shape = jax.ShapeDtypeStruct((4096, 4096), F32)
    o_shape = jax.ShapeDtypeStruct((2048, 2048), F32)
    j16 = lambda j: jnp.where(j < 32, j % 16, 15)
    return pl.pallas_call(
        functools.partial(_mmf_kernel, bj=bj),
        grid=(40,),
        in_specs=[
            pl.BlockSpec((1024, 4096), lambda j: (jnp.minimum(j // 16, 1), 0)),
            pl.BlockSpec((4096, bj), lambda j: (0, j16(j))),
            pl.BlockSpec((4096, bj), lambda j: (0, jnp.clip(j - 32, 0, 7))),
            pl.BlockSpec((1, bj), lambda j: (0, j16(j))),
            pl.BlockSpec((1, bj), lambda j: (0, j16(j))),
        ],
        out_specs=[
            pl.BlockSpec((2048, bj), lambda j: (jnp.minimum(j // 16, 1), j16(j))),
            pl.BlockSpec((2048, bj), lambda j: (jnp.minimum(j // 16, 1), j16(j))),
            pl.BlockSpec((2048, bj), lambda j: (0, jnp.clip(j - 32, 0, 7))),
        ],
        out_shape=[d_shape, d_shape, o_shape],
        scratch_shapes=[pltpu.VMEM((2048, 4096), BF16)],
        compiler_params=_cparams(1),
        name="mm_fused",
        interpret=_INTERPRET,
    )(A, B2, B3, z1, z2)


def kernel(x, W1, W2, W3):
    z1, W1T, W1Tb = _gemv_trans(x, W1, 512, relu=False, scale=False)
    z2, W2T, W2Tsb = _gemv_trans(z1, W2, 256, relu=True, scale=True)
    out, W3T, W3Tb, EYE = _gemv_trans(z2, W3, 256, relu=True, scale=False,
                                      eye=True)
    D1, D2, DJM = _mm_fused(W1Tb, W2Tsb, W3Tb, z1, z2)
    return (out, DJM, W1T, D1, W2T, D2, W3T, EYE)


# speedup vs baseline: 1.0030x; 1.0030x over previous
import functools

import jax
import jax.numpy as jnp
from jax.experimental import pallas as pl
from jax.experimental.pallas import tpu as pltpu

F32 = jnp.float32
BF16 = jnp.bfloat16
_VMEM_LIMIT = 63 * 1024 * 1024
_INTERPRET = False


def _cparams(n):
    return pltpu.CompilerParams(
        dimension_semantics=("arbitrary",) * n,
        vmem_limit_bytes=_VMEM_LIMIT,
    )


def _gemv_trans_kernel(h_ref, w_ref, z_ref, wt_ref, wtb_ref, *eye_ref,
                       relu, scale, eye_b):
    h = h_ref[...]
    if relu:
        h = jnp.maximum(h, 0.0)
    z_ref[...] = jax.lax.dot_general(
        h, w_ref[...], (((1,), (1,)), ((), ())),
        preferred_element_type=F32)
    w = w_ref[...]
    wt_ref[...] = w.T
    if scale:
        w = w * (h_ref[...] > 0).astype(F32)
    wtb_ref[...] = w.T.astype(BF16)
    if eye_b:
        jj = pl.program_id(0)
        rows = jax.lax.broadcasted_iota(jnp.int32, (2048, eye_b), 0)
        cols = jax.lax.broadcasted_iota(jnp.int32, (2048, eye_b), 1) + jj * eye_b
        eye_ref[0][...] = jnp.where(rows == cols, 1.0, 0.0).astype(F32)


def _gemv_trans(h, W, bj, relu, scale, eye=False):
    J, K = W.shape
    eye_b = (2048 * bj) // J if eye else 0
    out_specs = [pl.BlockSpec((1, bj), lambda j: (0, j)),
                 pl.BlockSpec((K, bj), lambda j: (0, j)),
                 pl.BlockSpec((K, bj), lambda j: (0, j))]
    out_shape = [jax.ShapeDtypeStruct((1, J), F32),
                 jax.ShapeDtypeStruct((K, J), F32),
                 jax.ShapeDtypeStruct((K, J), BF16)]
    if eye:
        out_specs.append(pl.BlockSpec((2048, eye_b), lambda j: (0, j)))
        out_shape.append(jax.ShapeDtypeStruct((2048, 2048), F32))
    return pl.pallas_call(
        functools.partial(_gemv_trans_kernel, relu=relu, scale=scale,
                          eye_b=eye_b),
        grid=(J // bj,),
        in_specs=[pl.BlockSpec((1, K), lambda j: (0, 0)),
                  pl.BlockSpec((bj, K), lambda j: (j, 0))],
        out_specs=out_specs,
        out_shape=out_shape,
        compiler_params=_cparams(1),
        name="gemv_trans",
        interpret=_INTERPRET,
    )(h, W)


def _l12_kernel(x_ref, w1_ref, w2_ref,
                z1_ref, w1t_ref, w1tb_ref,
                z2_ref, w2t_ref, w2tsb_ref, z1s_ref):
    j = pl.program_id(0)

    @pl.when(j < 16)
    def _():
        w = w1_ref[...]
        z = jax.lax.dot_general(x_ref[...], w, (((1,), (1,)), ((), ())),
                                preferred_element_type=F32)
        z1_ref[...] = z
        z1s_ref[0:1, pl.ds(pl.multiple_of(j * 256, 256), 256)] = z
        wt = w.T
        w1t_ref[...] = wt
        w1tb_ref[...] = wt.astype(BF16)

    @pl.when(j >= 16)
    def _():
        z1 = z1s_ref[...]
        h = jnp.maximum(z1, 0.0)
        w = w2_ref[...]
        z2_ref[...] = jax.lax.dot_general(h, w, (((1,), (1,)), ((), ())),
                                          preferred_element_type=F32)
        w2t_ref[...] = w.T
        w2tsb_ref[...] = (w * (z1 > 0).astype(F32)).T.astype(BF16)


def _l12(x, W1, W2):
    i1 = lambda j: jnp.minimum(j, 15)
    i2 = lambda j: jnp.clip(j - 16, 0, 7)
    return pl.pallas_call(
        _l12_kernel,
        grid=(24,),
        in_specs=[pl.BlockSpec((1, 2048), lambda j: (0, 0)),
                  pl.BlockSpec((256, 2048), lambda j: (i1(j), 0)),
                  pl.BlockSpec((512, 4096), lambda j: (i2(j), 0))],
        out_specs=[pl.BlockSpec((1, 256), lambda j: (0, i1(j))),
                   pl.BlockSpec((2048, 256), lambda j: (0, i1(j))),
                   pl.BlockSpec((2048, 256), lambda j: (0, i1(j))),
                   pl.BlockSpec((1, 512), lambda j: (0, i2(j))),
                   pl.BlockSpec((4096, 512), lambda j: (0, i2(j))),
                   pl.BlockSpec((4096, 512), lambda j: (0, i2(j)))],
        out_shape=[jax.ShapeDtypeStruct((1, 4096), F32),
                   jax.ShapeDtypeStruct((2048, 4096), F32),
                   jax.ShapeDtypeStruct((2048, 4096), BF16),
                   jax.ShapeDtypeStruct((1, 4096), F32),
                   jax.ShapeDtypeStruct((4096, 4096), F32),
                   jax.ShapeDtypeStruct((4096, 4096), BF16)],
        scratch_shapes=[pltpu.VMEM((1, 4096), F32)],
        compiler_params=_cparams(1),
        name="l12_fused",
        interpret=_INTERPRET,
    )(x, W1, W2)


def _mmf_kernel(a_ref, b2_ref, b3_ref, z1_ref, z2_ref,
                d1_ref, d2_ref, djm_ref, t1s_ref, *, bj):
    j = pl.program_id(0)

    @pl.when(j < 32)
    def _():
        i = j // 16
        jj = j - i * 16
        o = jnp.dot(a_ref[...], b2_ref[...], preferred_element_type=F32)
        m2 = (z2_ref[...] > 0).astype(F32)
        r0 = pl.multiple_of(i * 1024, 1024)
        c0 = pl.multiple_of(jj * bj, bj)
        t1s_ref[pl.ds(r0, 1024), pl.ds(c0, bj)] = (o * m2).astype(BF16)

        d1_ref[...] = jnp.zeros((2048, bj), F32)
        d2_ref[...] = jnp.zeros((2048, bj), F32)

        @pl.when(jj // 8 == i)
        def _():
            s0 = pl.multiple_of((jj - i * 8) * bj, bj)
            rs = jax.lax.broadcasted_iota(jnp.int32, (bj, bj), 0)
            cs = jax.lax.broadcasted_iota(jnp.int32, (bj, bj), 1)
            eqs = rs == cs
            d1_ref[pl.ds(s0, bj), :] = jnp.where(
                eqs, (z1_ref[...] > 0).astype(F32), 0.0)
            d2_ref[pl.ds(s0, bj), :] = jnp.where(eqs, m2, 0.0)

    @pl.when(j >= 32)
    def _():
        djm_ref[...] = jnp.dot(t1s_ref[...], b3_ref[...],
                               preferred_element_type=F32)


def _mm_fused(A, B2, B3, z1, z2, bj=256):
    d_shape = jax.ShapeDtypeStruct((4096, 4096), F32)
    o_shape = jax.ShapeDtypeStruct((2048, 2048), F32)
    j16 = lambda j: jnp.where(j < 32, j % 16, 15)
    return pl.pallas_call(
        functools.partial(_mmf_kernel, bj=bj),
        grid=(40,),
        in_specs=[
            pl.BlockSpec((1024, 4096), lambda j: (jnp.minimum(j // 16, 1), 0)),
            pl.BlockSpec((4096, bj), lambda j: (0, j16(j))),
            pl.BlockSpec((4096, bj), lambda j: (0, jnp.clip(j - 32, 0, 7))),
            pl.BlockSpec((1, bj), lambda j: (0, j16(j))),
            pl.BlockSpec((1, bj), lambda j: (0, j16(j))),
        ],
        out_specs=[
            pl.BlockSpec((2048, bj), lambda j: (jnp.minimum(j // 16, 1), j16(j))),
            pl.BlockSpec((2048, bj), lambda j: (jnp.minimum(j // 16, 1), j16(j))),
            pl.BlockSpec((2048, bj), lambda j: (0, jnp.clip(j - 32, 0, 7))),
        ],
        out_shape=[d_shape, d_shape, o_shape],
        scratch_shapes=[pltpu.VMEM((2048, 4096), BF16)],
        compiler_params=_cparams(1),
        name="mm_fused",
        interpret=_INTERPRET,
    )(A, B2, B3, z1, z2)


def kernel(x, W1, W2, W3):
    z1, W1T, W1Tb, z2, W2T, W2Tsb = _l12(x, W1, W2)
    out, W3T, W3Tb, EYE = _gemv_trans(z2, W3, 512, relu=True, scale=False,
                                      eye=True)
    D1, D2, DJM = _mm_fused(W1Tb, W2Tsb, W3Tb, z1, z2)
    return (out, DJM, W1T, D1, W2T, D2, W3T, EYE)


# speedup vs baseline: 1.0161x; 1.0130x over previous
import functools

import jax
import jax.numpy as jnp
from jax.experimental import pallas as pl
from jax.experimental.pallas import tpu as pltpu

F32 = jnp.float32
BF16 = jnp.bfloat16
_VMEM_LIMIT = 63 * 1024 * 1024
_INTERPRET = False


def _cparams(n):
    return pltpu.CompilerParams(
        dimension_semantics=("arbitrary",) * n,
        vmem_limit_bytes=_VMEM_LIMIT,
    )


def _gemv_trans_kernel(h_ref, w_ref, z_ref, wt_ref, wtb_ref, *eye_ref,
                       relu, scale, eye_b):
    h = h_ref[...]
    if relu:
        h = jnp.maximum(h, 0.0)
    z_ref[...] = jax.lax.dot_general(
        h, w_ref[...], (((1,), (1,)), ((), ())),
        preferred_element_type=F32)
    w = w_ref[...]
    wt_ref[...] = w.T
    if scale:
        w = w * (h_ref[...] > 0).astype(F32)
    wtb_ref[...] = w.T.astype(BF16)
    if eye_b:
        jj = pl.program_id(0)
        rows = jax.lax.broadcasted_iota(jnp.int32, (2048, eye_b), 0)
        cols = jax.lax.broadcasted_iota(jnp.int32, (2048, eye_b), 1) + jj * eye_b
        eye_ref[0][...] = jnp.where(rows == cols, 1.0, 0.0).astype(F32)


def _gemv_trans(h, W, bj, relu, scale, eye=False):
    J, K = W.shape
    eye_b = (2048 * bj) // J if eye else 0
    out_specs = [pl.BlockSpec((1, bj), lambda j: (0, j)),
                 pl.BlockSpec((K, bj), lambda j: (0, j)),
                 pl.BlockSpec((K, bj), lambda j: (0, j))]
    out_shape = [jax.ShapeDtypeStruct((1, J), F32),
                 jax.ShapeDtypeStruct((K, J), F32),
                 jax.ShapeDtypeStruct((K, J), BF16)]
    if eye:
        out_specs.append(pl.BlockSpec((2048, eye_b), lambda j: (0, j)))
        out_shape.append(jax.ShapeDtypeStruct((2048, 2048), F32))
    return pl.pallas_call(
        functools.partial(_gemv_trans_kernel, relu=relu, scale=scale,
                          eye_b=eye_b),
        grid=(J // bj,),
        in_specs=[pl.BlockSpec((1, K), lambda j: (0, 0)),
                  pl.BlockSpec((bj, K), lambda j: (j, 0))],
        out_specs=out_specs,
        out_shape=out_shape,
        compiler_params=_cparams(1),
        name="gemv_trans",
        interpret=_INTERPRET,
    )(h, W)


def _mmf_kernel(a_ref, b2_ref, b3_ref, z1_ref, z2_ref,
                d1_ref, d2_ref, djm_ref, t1s_ref, *, bj):
    j = pl.program_id(0)

    @pl.when(j < 32)
    def _():
        i = j // 16
        jj = j - i * 16
        o = jnp.dot(a_ref[...], b2_ref[...], preferred_element_type=F32)
        m2 = (z2_ref[...] > 0).astype(F32)
        r0 = pl.multiple_of(i * 1024, 1024)
        c0 = pl.multiple_of(jj * bj, bj)
        t1s_ref[pl.ds(r0, 1024), pl.ds(c0, bj)] = (o * m2).astype(BF16)

        d1_ref[...] = jnp.zeros((2048, bj), F32)
        d2_ref[...] = jnp.zeros((2048, bj), F32)

        @pl.when(jj // 8 == i)
        def _():
            s0 = pl.multiple_of((jj - i * 8) * bj, bj)
            rs = jax.lax.broadcasted_iota(jnp.int32, (bj, bj), 0)
            cs = jax.lax.broadcasted_iota(jnp.int32, (bj, bj), 1)
            eqs = rs == cs
            d1_ref[pl.ds(s0, bj), :] = jnp.where(
                eqs, (z1_ref[...] > 0).astype(F32), 0.0)
            d2_ref[pl.ds(s0, bj), :] = jnp.where(eqs, m2, 0.0)

    @pl.when(j >= 32)
    def _():
        djm_ref[...] = jnp.dot(t1s_ref[...], b3_ref[...],
                               preferred_element_type=F32)


def _mm_fused(A, B2, B3, z1, z2, bj=256):
    d_shape = jax.ShapeDtypeStruct((4096, 4096), F32)
    o_shape = jax.ShapeDtypeStruct((2048, 2048), F32)
    j16 = lambda j: jnp.where(j < 32, j % 16, 15)
    return pl.pallas_call(
        functools.partial(_mmf_kernel, bj=bj),
        grid=(40,),
        in_specs=[
            pl.BlockSpec((1024, 4096), lambda j: (jnp.minimum(j // 16, 1), 0)),
            pl.BlockSpec((4096, bj), lambda j: (0, j16(j))),
            pl.BlockSpec((4096, bj), lambda j: (0, jnp.clip(j - 32, 0, 7))),
            pl.BlockSpec((1, bj), lambda j: (0, j16(j))),
            pl.BlockSpec((1, bj), lambda j: (0, j16(j))),
        ],
        out_specs=[
            pl.BlockSpec((2048, bj), lambda j: (jnp.minimum(j // 16, 1), j16(j))),
            pl.BlockSpec((2048, bj), lambda j: (jnp.minimum(j // 16, 1), j16(j))),
            pl.BlockSpec((2048, bj), lambda j: (0, jnp.clip(j - 32, 0, 7))),
        ],
        out_shape=[d_shape, d_shape, o_shape],
        scratch_shapes=[pltpu.VMEM((2048, 4096), BF16)],
        compiler_params=_cparams(1),
        name="mm_fused",
        interpret=_INTERPRET,
    )(A, B2, B3, z1, z2)


def kernel(x, W1, W2, W3):
    z1, W1T, W1Tb = _gemv_trans(x, W1, 1024, relu=False, scale=False)
    z2, W2T, W2Tsb = _gemv_trans(z1, W2, 512, relu=True, scale=True)
    out, W3T, W3Tb, EYE = _gemv_trans(z2, W3, 512, relu=True, scale=False,
                                      eye=True)
    D1, D2, DJM = _mm_fused(W1Tb, W2Tsb, W3Tb, z1, z2)
    return (out, DJM, W1T, D1, W2T, D2, W3T, EYE)


# speedup vs baseline: 1.0205x; 1.0043x over previous
import functools

import jax
import jax.numpy as jnp
from jax.experimental import pallas as pl
from jax.experimental.pallas import tpu as pltpu

F32 = jnp.float32
BF16 = jnp.bfloat16
_VMEM_LIMIT = 63 * 1024 * 1024


def _cparams(n):
    return pltpu.CompilerParams(
        dimension_semantics=("arbitrary",) * n,
        vmem_limit_bytes=_VMEM_LIMIT,
    )


def _gemv_trans_kernel(h_ref, w_ref, z_ref, wt_ref, wtb_ref, *eye_ref,
                       relu, scale, eye_b):
    h = h_ref[...]
    if relu:
        h = jnp.maximum(h, 0.0)
    z_ref[...] = jax.lax.dot_general(
        h, w_ref[...], (((1,), (1,)), ((), ())),
        preferred_element_type=F32)
    w = w_ref[...]
    wt_ref[...] = w.T
    if scale:
        w = w * (h_ref[...] > 0).astype(F32)
    wtb_ref[...] = w.T.astype(BF16)
    if eye_b:
        jj = pl.program_id(0)
        rows = jax.lax.broadcasted_iota(jnp.int32, (2048, eye_b), 0)
        cols = jax.lax.broadcasted_iota(jnp.int32, (2048, eye_b), 1) + jj * eye_b
        eye_ref[0][...] = jnp.where(rows == cols, 1.0, 0.0).astype(F32)


def _gemv_trans(h, W, bj, relu, scale, eye=False):
    J, K = W.shape
    eye_b = (2048 * bj) // J if eye else 0
    out_specs = [pl.BlockSpec((1, bj), lambda j: (0, j)),
                 pl.BlockSpec((K, bj), lambda j: (0, j)),
                 pl.BlockSpec((K, bj), lambda j: (0, j))]
    out_shape = [jax.ShapeDtypeStruct((1, J), F32),
                 jax.ShapeDtypeStruct((K, J), F32),
                 jax.ShapeDtypeStruct((K, J), BF16)]
    if eye:
        out_specs.append(pl.BlockSpec((2048, eye_b), lambda j: (0, j)))
        out_shape.append(jax.ShapeDtypeStruct((2048, 2048), F32))
    return pl.pallas_call(
        functools.partial(_gemv_trans_kernel, relu=relu, scale=scale,
                          eye_b=eye_b),
        grid=(J // bj,),
        in_specs=[pl.BlockSpec((1, K), lambda j: (0, 0)),
                  pl.BlockSpec((bj, K), lambda j: (j, 0))],
        out_specs=out_specs,
        out_shape=out_shape,
        compiler_params=_cparams(1),
        name="gemv_trans",
    )(h, W)


def _mmf_kernel(a_ref, b2_ref, b3_ref, z1_ref, z2_ref,
                d1_ref, d2_ref, djm_ref, t1s_ref, *, bj):
    j = pl.program_id(0)

    @pl.when(j < 32)
    def _():
        i = j // 16
        jj = j - i * 16
        o = jnp.dot(a_ref[...], b2_ref[...], preferred_element_type=F32)
        m2 = (z2_ref[...] > 0).astype(F32)
        r0 = pl.multiple_of(i * 1024, 1024)
        c0 = pl.multiple_of(jj * bj, bj)
        t1s_ref[pl.ds(r0, 1024), pl.ds(c0, bj)] = (o * m2).astype(BF16)

        d1_ref[...] = jnp.zeros((2048, bj), F32)
        d2_ref[...] = jnp.zeros((2048, bj), F32)

        @pl.when(jj // 8 == i)
        def _():
            s0 = pl.multiple_of((jj - i * 8) * bj, bj)
            rs = jax.lax.broadcasted_iota(jnp.int32, (bj, bj), 0)
            cs = jax.lax.broadcasted_iota(jnp.int32, (bj, bj), 1)
            eqs = rs == cs
            d1_ref[pl.ds(s0, bj), :] = jnp.where(
                eqs, (z1_ref[...] > 0).astype(F32), 0.0)
            d2_ref[pl.ds(s0, bj), :] = jnp.where(eqs, m2, 0.0)

    @pl.when(j >= 32)
    def _():
        djm_ref[...] = jnp.dot(t1s_ref[...], b3_ref[...],
                               preferred_element_type=F32)


def _mm_fused(A, B2, B3, z1, z2, bj=256):
    d_shape = jax.ShapeDtypeStruct((4096, 4096), F32)
    o_shape = jax.ShapeDtypeStruct((2048, 2048), F32)
    j16 = lambda j: jnp.where(j < 32, j % 16, 15)
    return pl.pallas_call(
        functools.partial(_mmf_kernel, bj=bj),
        grid=(40,),
        in_specs=[
            pl.BlockSpec((1024, 4096), lambda j: (jnp.minimum(j // 16, 1), 0)),
            pl.BlockSpec((4096, bj), lambda j: (0, j16(j))),
            pl.BlockSpec((4096, bj), lambda j: (0, jnp.clip(j - 32, 0, 7))),
            pl.BlockSpec((1, bj), lambda j: (0, j16(j))),
            pl.BlockSpec((1, bj), lambda j: (0, j16(j))),
        ],
        out_specs=[
            pl.BlockSpec((2048, bj), lambda j: (jnp.minimum(j // 16, 1), j16(j))),
            pl.BlockSpec((2048, bj), lambda j: (jnp.minimum(j // 16, 1), j16(j))),
            pl.BlockSpec((2048, bj), lambda j: (0, jnp.clip(j - 32, 0, 7))),
        ],
        out_shape=[d_shape, d_shape, o_shape],
        scratch_shapes=[pltpu.VMEM((2048, 4096), BF16)],
        compiler_params=_cparams(1),
        name="mm_fused",
    )(A, B2, B3, z1, z2)


def kernel(x, W1, W2, W3):
    z1, W1T, W1Tb = _gemv_trans(x, W1, 1024, relu=False, scale=False)
    z2, W2T, W2Tsb = _gemv_trans(z1, W2, 512, relu=True, scale=True)
    out, W3T, W3Tb, EYE = _gemv_trans(z2, W3, 512, relu=True, scale=False,
                                      eye=True)
    D1, D2, DJM = _mm_fused(W1Tb, W2Tsb, W3Tb, z1, z2)
    return (out, DJM, W1T, D1, W2T, D2, W3T, EYE)
